# baseline (speedup 1.0000x reference)
import functools

import jax
import jax.numpy as jnp
from jax import lax
from jax.experimental import pallas as pl
from jax.experimental.pallas import tpu as pltpu
from jax.experimental.pallas import tpu_sc as plsc

N = 10000
E = 320000
D = 128

NC = 2
NS = 16
L = 16

CW = 125
NCH = E // CW
NCHT = NCH // (NC * NS)
IB = 16
ZCH = 80
NZ = N // ZCH
CL = 128


def _zero_rows(ref, nrows, width=D):
    zero16 = jnp.zeros((L,), jnp.float32)

    def row(i, carry):
        for l in range(width // L):
            ref[i, pl.ds(l * L, L)] = zero16
        return carry

    lax.fori_loop(0, nrows, row, 0)


def _round_robin_copy(sid, body_fn, nz=NZ, zch=ZCH):
    def step(j, carry):
        m = j * NS + sid

        @pl.when(m < nz)
        def _():
            body_fn(m * zch)
        return carry

    lax.fori_loop(0, (nz + NS - 1) // NS, step, 0)


def _make_seg_sum():
    mesh = plsc.VectorSubcoreMesh(
        core_axis_name="c", subcore_axis_name="s", num_cores=NC,
        num_subcores=NS)

    def body(table_hbm, src_hbm, dst_hbm, out_hbm,
             src_v, dst_v, rows_v, zrow_v, acc_sh,
             semg0, semg1, sems0, sems1):
        cid = lax.axis_index("c")
        sid = lax.axis_index("s")
        wid = sid * NC + cid

        _zero_rows(zrow_v, ZCH)
        _round_robin_copy(
            sid, lambda r0: pltpu.sync_copy(zrow_v, acc_sh.at[pl.ds(r0, ZCH)]))
        plsc.subcore_barrier()

        semg = (semg0, semg1)
        sems = (sems0, sems1)

        def start_gather(j, b):
            pltpu.async_copy(table_hbm.at[src_v.at[j]], rows_v.at[b], semg[b])

        def wait_gather(b):
            pltpu.make_async_copy(
                table_hbm.at[src_v.at[0]], rows_v.at[b], semg[b]).wait()

        def start_scatter(j, b):
            pltpu.async_copy(rows_v.at[b], acc_sh.at[dst_v.at[j]], sems[b],
                             add=True)

        def wait_scatter(b):
            pltpu.make_async_copy(
                rows_v.at[b], acc_sh.at[dst_v.at[0]], sems[b]).wait()

        def block(bi, carry):
            c0 = wid * NCHT + bi * IB
            pltpu.sync_copy(src_hbm.at[pl.ds(c0, IB)], src_v)
            pltpu.sync_copy(dst_hbm.at[pl.ds(c0, IB)], dst_v)

            start_gather(0, 0)

            def pair(jj, c):
                for b in range(2):
                    j = jj * 2 + b
                    wait_gather(b)
                    start_scatter(j, b)

                    @pl.when(j >= 1)
                    def _():
                        wait_scatter(1 - b)

                    @pl.when(j + 1 < IB)
                    def _():
                        start_gather(j + 1, 1 - b)
                return c

            lax.fori_loop(0, IB // 2, pair, 0)
            wait_scatter(1)
            return carry

        lax.fori_loop(0, NCHT // IB, block, 0)
        plsc.subcore_barrier()

        _round_robin_copy(
            sid, lambda r0: pltpu.sync_copy(acc_sh.at[pl.ds(r0, ZCH)],
                                            out_hbm.at[cid, pl.ds(r0, ZCH)]))

    return pl.kernel(
        body,
        out_type=(jax.ShapeDtypeStruct((NC, N, D), jnp.float32),),
        mesh=mesh,
        scratch_types=(
            pltpu.VMEM((IB, CW), jnp.int32),
            pltpu.VMEM((IB, CW), jnp.int32),
            pltpu.VMEM((2, CW, D), jnp.float32),
            pltpu.VMEM((ZCH, D), jnp.float32),
            pltpu.VMEM_SHARED((N, D), jnp.float32),
            pltpu.SemaphoreType.DMA,
            pltpu.SemaphoreType.DMA,
            pltpu.SemaphoreType.DMA,
            pltpu.SemaphoreType.DMA,
        ))


def _make_count():
    mesh = plsc.VectorSubcoreMesh(
        core_axis_name="c", subcore_axis_name="s", num_cores=NC,
        num_subcores=NS)

    def body(dst_hbm, out_hbm, dst_v, ones_v, zcnt_v, cnt_sh, sem0, sem1):
        cid = lax.axis_index("c")
        sid = lax.axis_index("s")
        wid = sid * NC + cid

        pltpu.sync_copy(dst_hbm.at[pl.ds(wid * NCHT, NCHT)], dst_v)

        one16 = jnp.full((L,), 1.0, jnp.float32)

        def fill_row(i, carry):
            for l in range(CL // L):
                @pl.when(i < ZCH)
                def _():
                    zcnt_v[i, pl.ds(l * L, L)] = jnp.zeros((L,), jnp.float32)
                ones_v[i, pl.ds(l * L, L)] = one16
            return carry

        lax.fori_loop(0, CW, fill_row, 0)
        _round_robin_copy(
            sid, lambda r0: pltpu.sync_copy(zcnt_v, cnt_sh.at[pl.ds(r0, ZCH)]))
        plsc.subcore_barrier()

        sems = (sem0, sem1)

        def pair(jj, carry):
            for b in range(2):
                j = jj * 2 + b

                @pl.when(j >= 2)
                def _():
                    pltpu.make_async_copy(
                        ones_v, cnt_sh.at[dst_v.at[0]], sems[b]).wait()

                pltpu.async_copy(ones_v, cnt_sh.at[dst_v.at[j]], sems[b],
                                 add=True)
            return carry

        lax.fori_loop(0, NCHT // 2, pair, 0)
        for b in range(2):
            pltpu.make_async_copy(
                ones_v, cnt_sh.at[dst_v.at[0]], sems[b]).wait()
        plsc.subcore_barrier()

        _round_robin_copy(
            sid, lambda r0: pltpu.sync_copy(cnt_sh.at[pl.ds(r0, ZCH)],
                                            out_hbm.at[cid, pl.ds(r0, ZCH)]))

    return pl.kernel(
        body,
        out_type=(jax.ShapeDtypeStruct((NC, N, CL), jnp.float32),),
        mesh=mesh,
        scratch_types=(
            pltpu.VMEM((NCHT, CW), jnp.int32),
            pltpu.VMEM((CW, CL), jnp.float32),
            pltpu.VMEM((ZCH, CL), jnp.float32),
            pltpu.VMEM_SHARED((N, CL), jnp.float32),
            pltpu.SemaphoreType.DMA,
            pltpu.SemaphoreType.DMA,
        ))


_seg_sum = _make_seg_sum()
_count = _make_count()


_RB = 1000


def _tc_pre_body(x_ref, w_ref, o_ref):
    o_ref[...] = lax.dot_general(
        x_ref[...], w_ref[...], (((1,), (1,)), ((), ())),
        preferred_element_type=jnp.float32,
        precision=lax.Precision.HIGHEST)


def _tc1_body(s_ref, c_ref, hr_ref, w1l_ref, b1_ref, w2l_ref, h_ref, p_ref):
    s = s_ref[0] + s_ref[1]
    cnt = c_ref[0][:, 0:1] + c_ref[1][:, 0:1]
    agg = s / jnp.maximum(cnt, 1.0)
    hp = lax.dot_general(agg, w1l_ref[...], (((1,), (1,)), ((), ())),
                         preferred_element_type=jnp.float32,
                         precision=lax.Precision.HIGHEST)
    h = jnp.maximum(hp + b1_ref[...] + hr_ref[...], 0.0)
    h_ref[...] = h
    p_ref[...] = lax.dot_general(h, w2l_ref[...], (((1,), (1,)), ((), ())),
                                 preferred_element_type=jnp.float32,
                                 precision=lax.Precision.HIGHEST)


def _tc_hr2_body(h_ref, w2r_ref, b2_ref, o_ref):
    o_ref[...] = b2_ref[...] + lax.dot_general(
        h_ref[...], w2r_ref[...], (((1,), (1,)), ((), ())),
        preferred_element_type=jnp.float32,
        precision=lax.Precision.HIGHEST)


def _tc2_body(s_ref, c_ref, hr2_ref, o_ref):
    s = s_ref[0] + s_ref[1]
    cnt = c_ref[0][:, 0:1] + c_ref[1][:, 0:1]
    agg = s / jnp.maximum(cnt, 1.0)
    o_ref[...] = agg + hr2_ref[...]


def kernel(x, edge_index, W1l, b1, W1r, W2l, b2, W2r):
    H = W1l.shape[0]
    O = W2l.shape[0]
    src = edge_index[0].reshape(NCH, CW)
    dst = edge_index[1].reshape(NCH, CW)

    grid = (N // _RB,)

    hr1 = pl.pallas_call(
        _tc_pre_body,
        grid=grid,
        in_specs=[
            pl.BlockSpec((_RB, D), lambda i: (i, 0)),
            pl.BlockSpec((H, D), lambda i: (0, 0)),
        ],
        out_specs=pl.BlockSpec((_RB, H), lambda i: (i, 0)),
        out_shape=jax.ShapeDtypeStruct((N, H), jnp.float32),
    )(x, W1r)

    (cnt,) = _count(dst)
    (s1,) = _seg_sum(x, src, dst)

    h, p = pl.pallas_call(
        _tc1_body,
        grid=grid,
        in_specs=[
            pl.BlockSpec((NC, _RB, D), lambda i: (0, i, 0)),
            pl.BlockSpec((NC, _RB, CL), lambda i: (0, i, 0)),
            pl.BlockSpec((_RB, H), lambda i: (i, 0)),
            pl.BlockSpec((H, D), lambda i: (0, 0)),
            pl.BlockSpec((1, H), lambda i: (0, 0)),
            pl.BlockSpec((O, H), lambda i: (0, 0)),
        ],
        out_specs=[
            pl.BlockSpec((_RB, H), lambda i: (i, 0)),
            pl.BlockSpec((_RB, O), lambda i: (i, 0)),
        ],
        out_shape=[
            jax.ShapeDtypeStruct((N, H), jnp.float32),
            jax.ShapeDtypeStruct((N, O), jnp.float32),
        ],
    )(s1, cnt, hr1, W1l, b1.reshape(1, H), W2l)

    (s2,) = _seg_sum(p, src, dst)

    hr2 = pl.pallas_call(
        _tc_hr2_body,
        grid=grid,
        in_specs=[
            pl.BlockSpec((_RB, H), lambda i: (i, 0)),
            pl.BlockSpec((O, H), lambda i: (0, 0)),
            pl.BlockSpec((1, O), lambda i: (0, 0)),
        ],
        out_specs=pl.BlockSpec((_RB, O), lambda i: (i, 0)),
        out_shape=jax.ShapeDtypeStruct((N, O), jnp.float32),
    )(h, W2r, b2.reshape(1, O))

    out = pl.pallas_call(
        _tc2_body,
        grid=grid,
        in_specs=[
            pl.BlockSpec((NC, _RB, O), lambda i: (0, i, 0)),
            pl.BlockSpec((NC, _RB, CL), lambda i: (0, i, 0)),
            pl.BlockSpec((_RB, O), lambda i: (i, 0)),
        ],
        out_specs=pl.BlockSpec((_RB, O), lambda i: (i, 0)),
        out_shape=jax.ShapeDtypeStruct((N, O), jnp.float32),
    )(s2, cnt, hr2)
    return out
